# baseline (speedup 1.0000x reference)
import jax
import jax.numpy as jnp
from jax import lax
from jax.experimental import pallas as pl
from jax.experimental.pallas import tpu as pltpu
from jax.experimental.pallas import tpu_sc as plsc

E = 16384
H = 128
NC = 2
NS = 16
NW = NC * NS
EPW = E // NW
CHUNK = 128
NCH = EPW // CHUNK
L = 16
GPC = CHUNK // L
HC = H // L


def _rsqrt(x):
    i = plsc.bitcast(x, jnp.int32)
    i = jnp.int32(0x5F3759DF) - (i >> 1)
    y = plsc.bitcast(i, jnp.float32)
    for _ in range(2):
        y = y * (1.5 - 0.5 * x * y * y)
    return y


def _body(emb_hbm, ei_hbm, d_hbm, scale_hbm, out_hbm,
          u_idx, v_idx, d_v, scale_v, eu0, ev0, eu1, ev1, eu2, ev2,
          tdot, tsu, tsv, out_v, su0, sv0, su1, sv1, su2, sv2):
    cid = lax.axis_index("c")
    sid = lax.axis_index("s")
    wid = sid * NC + cid
    base = wid * EPW

    pltpu.sync_copy(ei_hbm.at[0, pl.ds(base, CHUNK)], u_idx.at[pl.ds(0, CHUNK)])
    pltpu.sync_copy(ei_hbm.at[1, pl.ds(base, CHUNK)], v_idx.at[pl.ds(0, CHUNK)])

    eus = [eu0, eu1, eu2]
    evs = [ev0, ev1, ev2]
    sems_u = [su0, su1, su2]
    sems_v = [sv0, sv1, sv2]

    def start(j):
        b = j % 3
        cu = pltpu.async_copy(
            emb_hbm.at[u_idx.at[pl.ds(j * CHUNK, CHUNK)]], eus[b], sems_u[b])
        cv = pltpu.async_copy(
            emb_hbm.at[v_idx.at[pl.ds(j * CHUNK, CHUNK)]], evs[b], sems_v[b])
        return cu, cv

    pends = {0: start(0)}

    pltpu.sync_copy(ei_hbm.at[0, pl.ds(base + CHUNK, EPW - CHUNK)],
                    u_idx.at[pl.ds(CHUNK, EPW - CHUNK)])
    pltpu.sync_copy(ei_hbm.at[1, pl.ds(base + CHUNK, EPW - CHUNK)],
                    v_idx.at[pl.ds(CHUNK, EPW - CHUNK)])
    pltpu.sync_copy(d_hbm, d_v)
    pltpu.sync_copy(scale_hbm, scale_v)
    zeros16 = jnp.zeros((L,), jnp.int32)
    scv = plsc.load_gather(scale_v, [zeros16])
    dreg = [d_v[pl.ds(c * L, L)] * scv for c in range(HC)]
    tcol = lax.iota(jnp.int32, L) * L
    for j in range(NCH):
        pends[j][0].wait()
        pends[j][1].wait()
        if j + 1 < NCH and (j + 1) not in pends:
            pends[j + 1] = start(j + 1)
        eu = eus[j % 3]
        ev = evs[j % 3]

        def group(g, _):
            @plsc.parallel_loop(0, L)
            def edge(el):
                e = g * L + el
                dot = None
                su = None
                sv = None
                for c in range(HC):
                    a = eu[e, pl.ds(c * L, L)]
                    b = ev[e, pl.ds(c * L, L)]
                    t = a * b
                    if c == 0:
                        dot = t * dreg[c]
                        su = a * a
                        sv = b * b
                    else:
                        dot = dot + t * dreg[c]
                        su = su + a * a
                        sv = sv + b * b
                slot = tcol + el
                plsc.store_scatter(tdot, [slot], dot)
                plsc.store_scatter(tsu, [slot], su)
                plsc.store_scatter(tsv, [slot], sv)

            def colsum(t):
                cols = [t[pl.ds(c * L, L)] for c in range(L)]
                while len(cols) > 1:
                    cols = [cols[i] + cols[i + 1]
                            for i in range(0, len(cols), 2)]
                return cols[0]

            dotv = colsum(tdot)
            suv = colsum(tsu)
            svv = colsum(tsv)
            res = dotv * _rsqrt(suv) * _rsqrt(svv)
            out_v[pl.ds(j * CHUNK + g * L, L)] = res
            return 0

        lax.fori_loop(0, GPC, group, 0)
        if j + 2 < NCH:
            pends[j + 2] = start(j + 2)

    pltpu.sync_copy(out_v, out_hbm.at[pl.ds(base, EPW)])


@jax.jit
def kernel(emb, edge_index, d, scale):
    mesh = plsc.VectorSubcoreMesh(core_axis_name="c", subcore_axis_name="s")
    run = pl.kernel(
        _body,
        mesh=mesh,
        compiler_params=pltpu.CompilerParams(
            needs_layout_passes=False,
            disable_bounds_checks=True,
            disable_semaphore_checks=True,
            skip_device_barrier=True,
        ),
        out_type=jax.ShapeDtypeStruct((E,), jnp.float32),
        scratch_types=[
            pltpu.VMEM((EPW,), jnp.int32),
            pltpu.VMEM((EPW,), jnp.int32),
            pltpu.VMEM((H,), jnp.float32),
            pltpu.VMEM((1,), jnp.float32),
            pltpu.VMEM((CHUNK, H), jnp.float32),
            pltpu.VMEM((CHUNK, H), jnp.float32),
            pltpu.VMEM((CHUNK, H), jnp.float32),
            pltpu.VMEM((CHUNK, H), jnp.float32),
            pltpu.VMEM((CHUNK, H), jnp.float32),
            pltpu.VMEM((CHUNK, H), jnp.float32),
            pltpu.VMEM((L * L,), jnp.float32),
            pltpu.VMEM((L * L,), jnp.float32),
            pltpu.VMEM((L * L,), jnp.float32),
            pltpu.VMEM((EPW,), jnp.float32),
            pltpu.SemaphoreType.DMA,
            pltpu.SemaphoreType.DMA,
            pltpu.SemaphoreType.DMA,
            pltpu.SemaphoreType.DMA,
            pltpu.SemaphoreType.DMA,
            pltpu.SemaphoreType.DMA,
        ],
    )
    return run(emb, edge_index.astype(jnp.int32), d.astype(jnp.float32),
               scale.astype(jnp.float32))


# speedup vs baseline: 1.0131x; 1.0131x over previous
import jax
import jax.numpy as jnp
from jax import lax
from jax.experimental import pallas as pl
from jax.experimental.pallas import tpu as pltpu
from jax.experimental.pallas import tpu_sc as plsc

E = 16384
H = 128
NC = 2
NS = 16
NW = NC * NS
EPW = E // NW
CHUNK = 128
NCH = EPW // CHUNK
L = 16
GPC = CHUNK // L
HC = H // L


def _rsqrt(x):
    i = plsc.bitcast(x, jnp.int32)
    i = jnp.int32(0x5F3759DF) - (i >> 1)
    y = plsc.bitcast(i, jnp.float32)
    for _ in range(2):
        y = y * (1.5 - 0.5 * x * y * y)
    return y


def _body(emb_hbm, ei_hbm, d_hbm, scale_hbm, out_hbm,
          u_idx, v_idx, d_v, scale_v, eu0, ev0, eu1, ev1, eu2, ev2,
          tdot, tsu, tsv, out_v, su0, sv0, su1, sv1, su2, sv2):
    cid = lax.axis_index("c")
    sid = lax.axis_index("s")
    wid = sid * NC + cid
    base = wid * EPW

    pltpu.sync_copy(ei_hbm.at[0, pl.ds(base, EPW)], u_idx)
    pltpu.sync_copy(ei_hbm.at[1, pl.ds(base, EPW)], v_idx)

    eus = [eu0, eu1, eu2]
    evs = [ev0, ev1, ev2]
    sems_u = [su0, su1, su2]
    sems_v = [sv0, sv1, sv2]

    def start(j):
        b = j % 3
        cu = pltpu.async_copy(
            emb_hbm.at[u_idx.at[pl.ds(j * CHUNK, CHUNK)]], eus[b], sems_u[b])
        cv = pltpu.async_copy(
            emb_hbm.at[v_idx.at[pl.ds(j * CHUNK, CHUNK)]], evs[b], sems_v[b])
        return cu, cv

    pends = {0: start(0)}

    pltpu.sync_copy(d_hbm, d_v)
    pltpu.sync_copy(scale_hbm, scale_v)
    zeros16 = jnp.zeros((L,), jnp.int32)
    scv = plsc.load_gather(scale_v, [zeros16])
    dreg = [d_v[pl.ds(c * L, L)] * scv for c in range(HC)]
    tcol = lax.iota(jnp.int32, L) * L
    for j in range(NCH):
        pends[j][0].wait()
        pends[j][1].wait()
        if j + 1 < NCH and (j + 1) not in pends:
            pends[j + 1] = start(j + 1)
        eu = eus[j % 3]
        ev = evs[j % 3]

        def group(g, _):
            @plsc.parallel_loop(0, L)
            def edge(el):
                e = g * L + el
                dot = None
                su = None
                sv = None
                for c in range(HC):
                    a = eu[e, pl.ds(c * L, L)]
                    b = ev[e, pl.ds(c * L, L)]
                    t = a * b
                    if c == 0:
                        dot = t * dreg[c]
                        su = a * a
                        sv = b * b
                    else:
                        dot = dot + t * dreg[c]
                        su = su + a * a
                        sv = sv + b * b
                slot = tcol + el
                plsc.store_scatter(tdot, [slot], dot)
                plsc.store_scatter(tsu, [slot], su)
                plsc.store_scatter(tsv, [slot], sv)

            def colsum(t):
                cols = [t[pl.ds(c * L, L)] for c in range(L)]
                while len(cols) > 1:
                    cols = [cols[i] + cols[i + 1]
                            for i in range(0, len(cols), 2)]
                return cols[0]

            dotv = colsum(tdot)
            suv = colsum(tsu)
            svv = colsum(tsv)
            res = dotv * _rsqrt(suv) * _rsqrt(svv)
            out_v[pl.ds(j * CHUNK + g * L, L)] = res
            return 0

        lax.fori_loop(0, GPC, group, 0)
        if j + 2 < NCH:
            pends[j + 2] = start(j + 2)

    pltpu.sync_copy(out_v, out_hbm.at[pl.ds(base, EPW)])


@jax.jit
def kernel(emb, edge_index, d, scale):
    mesh = plsc.VectorSubcoreMesh(core_axis_name="c", subcore_axis_name="s")
    run = pl.kernel(
        _body,
        mesh=mesh,
        compiler_params=pltpu.CompilerParams(
            needs_layout_passes=False,
            disable_bounds_checks=True,
            disable_semaphore_checks=True,
            skip_device_barrier=True,
        ),
        out_type=jax.ShapeDtypeStruct((E,), jnp.float32),
        scratch_types=[
            pltpu.VMEM((EPW,), jnp.int32),
            pltpu.VMEM((EPW,), jnp.int32),
            pltpu.VMEM((H,), jnp.float32),
            pltpu.VMEM((1,), jnp.float32),
            pltpu.VMEM((CHUNK, H), jnp.float32),
            pltpu.VMEM((CHUNK, H), jnp.float32),
            pltpu.VMEM((CHUNK, H), jnp.float32),
            pltpu.VMEM((CHUNK, H), jnp.float32),
            pltpu.VMEM((CHUNK, H), jnp.float32),
            pltpu.VMEM((CHUNK, H), jnp.float32),
            pltpu.VMEM((L * L,), jnp.float32),
            pltpu.VMEM((L * L,), jnp.float32),
            pltpu.VMEM((L * L,), jnp.float32),
            pltpu.VMEM((EPW,), jnp.float32),
            pltpu.SemaphoreType.DMA,
            pltpu.SemaphoreType.DMA,
            pltpu.SemaphoreType.DMA,
            pltpu.SemaphoreType.DMA,
            pltpu.SemaphoreType.DMA,
            pltpu.SemaphoreType.DMA,
        ],
    )
    return run(emb, edge_index.astype(jnp.int32), d.astype(jnp.float32),
               scale.astype(jnp.float32))


# speedup vs baseline: 1.0141x; 1.0009x over previous
import jax
import jax.numpy as jnp
from jax import lax
from jax.experimental import pallas as pl
from jax.experimental.pallas import tpu as pltpu
from jax.experimental.pallas import tpu_sc as plsc

E = 16384
H = 128
NC = 2
NS = 16
NW = NC * NS
EPW = E // NW
CHUNK = 128
NCH = EPW // CHUNK
L = 16
GPC = CHUNK // L
HC = H // L


def _rsqrt(x):
    i = plsc.bitcast(x, jnp.int32)
    i = jnp.int32(0x5F3759DF) - (i >> 1)
    y = plsc.bitcast(i, jnp.float32)
    for _ in range(2):
        y = y * (1.5 - 0.5 * x * y * y)
    return y


def _body(emb_hbm, ei_hbm, d_hbm, scale_hbm, out_hbm,
          u_idx, v_idx, d_v, scale_v, eu0, ev0, eu1, ev1, eu2, ev2,
          tdot, tsu, tsv, out_v, su0, sv0, su1, sv1, su2, sv2):
    cid = lax.axis_index("c")
    sid = lax.axis_index("s")
    wid = sid * NC + cid
    base = wid * EPW

    pltpu.sync_copy(ei_hbm.at[0, pl.ds(base, EPW)], u_idx)
    pltpu.sync_copy(ei_hbm.at[1, pl.ds(base, EPW)], v_idx)

    eus = [eu0, eu1, eu2]
    evs = [ev0, ev1, ev2]
    sems_u = [su0, su1, su2]
    sems_v = [sv0, sv1, sv2]

    def start(j):
        b = j % 3
        cu = pltpu.async_copy(
            emb_hbm.at[u_idx.at[pl.ds(j * CHUNK, CHUNK)]], eus[b], sems_u[b])
        cv = pltpu.async_copy(
            emb_hbm.at[v_idx.at[pl.ds(j * CHUNK, CHUNK)]], evs[b], sems_v[b])
        return cu, cv

    pends = {0: start(0)}

    pltpu.sync_copy(d_hbm, d_v)
    pltpu.sync_copy(scale_hbm, scale_v)
    zeros16 = jnp.zeros((L,), jnp.int32)
    scv = plsc.load_gather(scale_v, [zeros16])
    dreg = [d_v[pl.ds(c * L, L)] * scv for c in range(HC)]
    tcol = lax.iota(jnp.int32, L) * L
    for j in range(NCH):
        pends[j][0].wait()
        pends[j][1].wait()
        if j + 1 < NCH and (j + 1) not in pends:
            pends[j + 1] = start(j + 1)
        eu = eus[j % 3]
        ev = evs[j % 3]

        def group(g, _):
            @plsc.parallel_loop(0, L)
            def edge(el):
                e = g * L + el
                dot = None
                su = None
                sv = None
                for c in range(HC):
                    a = eu[e, pl.ds(c * L, L)]
                    b = ev[e, pl.ds(c * L, L)]
                    t = a * b
                    if c == 0:
                        dot = t * dreg[c]
                        su = a * a
                        sv = b * b
                    else:
                        dot = dot + t * dreg[c]
                        su = su + a * a
                        sv = sv + b * b
                slot = tcol + el
                plsc.store_scatter(tdot, [slot], dot)
                plsc.store_scatter(tsu, [slot], su)
                plsc.store_scatter(tsv, [slot], sv)

            def colsum(t):
                cols = [t[pl.ds(c * L, L)] for c in range(L)]
                while len(cols) > 1:
                    cols = [cols[i] + cols[i + 1]
                            for i in range(0, len(cols), 2)]
                return cols[0]

            dotv = colsum(tdot)
            suv = colsum(tsu)
            svv = colsum(tsv)
            res = dotv * _rsqrt(suv) * _rsqrt(svv)
            out_v[pl.ds(j * CHUNK + g * L, L)] = res
            return 0

        lax.fori_loop(0, GPC, group, 0)
        if j + 2 < NCH:
            pends[j + 2] = start(j + 2)

    pltpu.sync_copy(out_v, out_hbm.at[pl.ds(base, EPW)])


@jax.jit
def kernel(emb, edge_index, d, scale):
    mesh = plsc.VectorSubcoreMesh(core_axis_name="c", subcore_axis_name="s")
    run = pl.kernel(
        _body,
        mesh=mesh,
        compiler_params=pltpu.CompilerParams(
            needs_layout_passes=False,
            disable_bounds_checks=True,
            disable_semaphore_checks=True,
        ),
        out_type=jax.ShapeDtypeStruct((E,), jnp.float32),
        scratch_types=[
            pltpu.VMEM((EPW,), jnp.int32),
            pltpu.VMEM((EPW,), jnp.int32),
            pltpu.VMEM((H,), jnp.float32),
            pltpu.VMEM((1,), jnp.float32),
            pltpu.VMEM((CHUNK, H), jnp.float32),
            pltpu.VMEM((CHUNK, H), jnp.float32),
            pltpu.VMEM((CHUNK, H), jnp.float32),
            pltpu.VMEM((CHUNK, H), jnp.float32),
            pltpu.VMEM((CHUNK, H), jnp.float32),
            pltpu.VMEM((CHUNK, H), jnp.float32),
            pltpu.VMEM((L * L,), jnp.float32),
            pltpu.VMEM((L * L,), jnp.float32),
            pltpu.VMEM((L * L,), jnp.float32),
            pltpu.VMEM((EPW,), jnp.float32),
            pltpu.SemaphoreType.DMA,
            pltpu.SemaphoreType.DMA,
            pltpu.SemaphoreType.DMA,
            pltpu.SemaphoreType.DMA,
            pltpu.SemaphoreType.DMA,
            pltpu.SemaphoreType.DMA,
        ],
    )
    return run(emb, edge_index.astype(jnp.int32), d.astype(jnp.float32),
               scale.astype(jnp.float32))
